# baseline (speedup 1.0000x reference)
import functools

import jax
import jax.numpy as jnp
from jax.experimental import pallas as pl
from jax.experimental.pallas import tpu as pltpu

_BF = jnp.bfloat16

_PAR = (1, 0, 1, 0)
_OFF = (0, 1, 1, 2)


def _pack_patch_weights(wg, cin, cout):
    w = wg.reshape(3, 3, cin, cout)
    phases = [
        jnp.pad(w, ((py, 1 - py), (px, 1 - px), (0, 0), (0, 0)))
        for py in range(2) for px in range(2)
    ]
    wp = jnp.stack(phases, axis=3)
    return wp.reshape(4, 4 * cin, 4 * cout)


def _row_windows(y, dtype):
    b, h, w, c = y.shape
    ws = w // 2
    yw = jnp.pad(y, ((0, 0), (0, 0), (1, 1), (0, 0)))
    lo = yw[:, :, 0:w, :].reshape(b, h, ws, 2 * c)
    hi = yw[:, :, 2:w + 2, :].reshape(b, h, ws, 2 * c)
    win = jnp.concatenate([lo, hi], axis=-1).astype(dtype)
    return [
        jnp.pad(win[:, r::2], ((0, 0), (1, 1), (0, 0), (0, 0)))
        for r in range(2)
    ]


def _l1_kernel(a_ref, w_ref, b_ref, o_ref):
    z = jnp.dot(a_ref[...], w_ref[...],
                preferred_element_type=jnp.float32).astype(_BF)
    m = jnp.maximum(jnp.maximum(z[:, :32], z[:, 32:64]),
                    jnp.maximum(z[:, 64:96], z[:, 96:]))
    o_ref[...] = jnp.maximum(m + b_ref[...], 0.0)


def _conv_win_kernel(r0_ref, r1_ref, w_ref, b_ref, o_ref, *, hs, ws, cout,
                     max_dtype, out_dtype):
    rs = (r0_ref, r1_ref)
    taps = []
    for sy in range(4):
        a = rs[_PAR[sy]][:, _OFF[sy]:_OFF[sy] + hs, :, :]
        a = a.astype(w_ref.dtype)
        taps.append(a.reshape(-1, a.shape[-1]))
    a = jnp.concatenate(taps, axis=-1)
    w = w_ref[...].reshape(-1, w_ref.shape[-1])
    z = jnp.dot(a, w, preferred_element_type=jnp.float32)
    z = z.astype(max_dtype)
    m = jnp.maximum(
        jnp.maximum(z[:, :cout], z[:, cout:2 * cout]),
        jnp.maximum(z[:, 2 * cout:3 * cout], z[:, 3 * cout:]),
    )
    y = jnp.maximum(m + b_ref[...].astype(max_dtype), 0.0).astype(out_dtype)
    o_ref[...] = y.reshape(o_ref.shape)


def _fc_kernel(a_ref, w1_ref, b1_ref, w2_ref, b2_ref, o_ref):
    h = jnp.dot(a_ref[...], w1_ref[...], preferred_element_type=jnp.float32)
    h = jnp.maximum(h + b1_ref[...], 0.0).astype(_BF)
    o_ref[...] = (jnp.dot(h, w2_ref[...], preferred_element_type=jnp.float32)
                  + b2_ref[...])


def _conv_layer(rwins, wp, b, tb, hs, ws, cin, cout, max_dtype, out_dtype):
    bsz = rwins[0].shape[0]
    tb = min(tb, bsz)
    dt = rwins[0].dtype
    return pl.pallas_call(
        functools.partial(_conv_win_kernel, hs=hs, ws=ws, cout=cout,
                          max_dtype=max_dtype, out_dtype=out_dtype),
        out_shape=jax.ShapeDtypeStruct((bsz, hs, ws, cout), out_dtype),
        grid=(bsz // tb,),
        in_specs=[pl.BlockSpec((tb, hs + 2, ws, 4 * cin), lambda i: (i, 0, 0, 0))
                  for _ in range(2)]
                 + [pl.BlockSpec((4, 4 * cin, 4 * cout), lambda i: (0, 0, 0)),
                    pl.BlockSpec((1, cout), lambda i: (0, 0))],
        out_specs=pl.BlockSpec((tb, hs, ws, cout), lambda i: (i, 0, 0, 0)),
        compiler_params=pltpu.CompilerParams(
            dimension_semantics=("arbitrary",)),
    )(rwins[0].astype(dt), rwins[1].astype(dt), wp, b)


def kernel(x_nchw, w1g, b1, w2g, b2, w3g, b3, fc1_wg, fc1_b, fc2_wg, fc2_b):
    bsz = x_nchw.shape[0]
    x = x_nchw.reshape(bsz, 64, 64)

    xp = jnp.pad(x, ((0, 0), (1, 1), (1, 1))).astype(_BF)
    w4 = jnp.concatenate([xp[:, :, 0:64].reshape(bsz, 66, 32, 2),
                          xp[:, :, 2:66].reshape(bsz, 66, 32, 2)],
                         axis=-1)
    h4 = jnp.concatenate([w4[:, 0:64].reshape(bsz, 32, 2, 32, 4),
                          w4[:, 2:66].reshape(bsz, 32, 2, 32, 4)],
                         axis=2)
    x16 = h4.transpose(0, 1, 3, 2, 4).reshape(bsz * 1024, 16)
    wp1 = _pack_patch_weights(w1g, 1, 32).reshape(16, 128).astype(_BF)
    b1b = b1.astype(_BF)

    m1 = bsz * 1024
    tm = min(32768, m1)
    y1 = pl.pallas_call(
        _l1_kernel,
        out_shape=jax.ShapeDtypeStruct((m1, 32), _BF),
        grid=(m1 // tm,),
        in_specs=[pl.BlockSpec((tm, 16), lambda i: (i, 0)),
                  pl.BlockSpec((16, 128), lambda i: (0, 0)),
                  pl.BlockSpec((1, 32), lambda i: (0, 0))],
        out_specs=pl.BlockSpec((tm, 32), lambda i: (i, 0)),
        compiler_params=pltpu.CompilerParams(
            dimension_semantics=("arbitrary",)),
    )(x16, wp1, b1b)
    y1 = y1.reshape(bsz, 32, 32, 32)

    wp2 = _pack_patch_weights(w2g, 32, 64).astype(_BF)
    y2 = _conv_layer(_row_windows(y1, _BF), wp2, b2, 64, 16, 16, 32, 64,
                     _BF, _BF)

    wp3 = _pack_patch_weights(w3g, 64, 128)
    y3 = _conv_layer(_row_windows(y2, _BF), wp3, b3, 64, 8, 8, 64, 128,
                     jnp.float32, _BF)

    feat = y3.reshape(bsz, 8 * 8 * 128)
    nc = fc2_wg.shape[1]
    ncp = max(128, ((nc + 127) // 128) * 128)
    w1f = fc1_wg.astype(_BF)
    w2f = jnp.pad(fc2_wg, ((0, 0), (0, ncp - nc))).astype(_BF)
    b2f = jnp.pad(fc2_b, ((0, 0), (0, ncp - nc)))
    tmf = min(256, bsz)
    out = pl.pallas_call(
        _fc_kernel,
        out_shape=jax.ShapeDtypeStruct((bsz, ncp), jnp.float32),
        grid=(bsz // tmf,),
        in_specs=[pl.BlockSpec((tmf, 8192), lambda i: (i, 0)),
                  pl.BlockSpec((8192, 512), lambda i: (0, 0)),
                  pl.BlockSpec((1, 512), lambda i: (0, 0)),
                  pl.BlockSpec((512, ncp), lambda i: (0, 0)),
                  pl.BlockSpec((1, ncp), lambda i: (0, 0))],
        out_specs=pl.BlockSpec((tmf, ncp), lambda i: (i, 0)),
        compiler_params=pltpu.CompilerParams(
            dimension_semantics=("arbitrary",)),
    )(feat, w1f, fc1_b, w2f, b2f)
    return out[:, :nc]


# speedup vs baseline: 1.3016x; 1.3016x over previous
import functools

import jax
import jax.numpy as jnp
from jax.experimental import pallas as pl
from jax.experimental.pallas import tpu as pltpu

_BF = jnp.bfloat16

_PAR = (1, 0, 1, 0)
_OFF = (0, 1, 1, 2)


def _pack_patch_weights(wg, cin, cout):
    w = wg.reshape(3, 3, cin, cout)
    phases = [
        jnp.pad(w, ((py, 1 - py), (px, 1 - px), (0, 0), (0, 0)))
        for py in range(2) for px in range(2)
    ]
    wp = jnp.stack(phases, axis=3)
    return wp.reshape(4, 4 * cin, 4 * cout)


def _row_windows(y, dtype):
    b, h, w, c = y.shape
    ws = w // 2
    yw = jnp.pad(y, ((0, 0), (0, 0), (1, 1), (0, 0)))
    lo = yw[:, :, 0:w, :].reshape(b, h, ws, 2 * c)
    hi = yw[:, :, 2:w + 2, :].reshape(b, h, ws, 2 * c)
    win = jnp.concatenate([lo, hi], axis=-1).astype(dtype)
    return [
        jnp.pad(win[:, r::2], ((0, 0), (1, 1), (0, 0), (0, 0)))
        for r in range(2)
    ]


def _l1_kernel(a_ref, w_ref, b_ref, o_ref):
    z = jnp.dot(a_ref[...], w_ref[...],
                preferred_element_type=jnp.float32).astype(_BF)
    m = jnp.maximum(jnp.maximum(z[:, :32], z[:, 32:64]),
                    jnp.maximum(z[:, 64:96], z[:, 96:]))
    o_ref[...] = jnp.maximum(m + b_ref[...], 0.0)


def _conv_win_kernel(r0_ref, r1_ref, w_ref, b_ref, o_ref, *, hs, ws, cout,
                     max_dtype, out_dtype):
    rs = (r0_ref, r1_ref)
    taps = []
    for sy in range(4):
        a = rs[_PAR[sy]][:, _OFF[sy]:_OFF[sy] + hs, :, :]
        a = a.astype(w_ref.dtype)
        taps.append(a.reshape(-1, a.shape[-1]))
    a = jnp.concatenate(taps, axis=-1)
    w = w_ref[...].reshape(-1, w_ref.shape[-1])
    z = jnp.dot(a, w, preferred_element_type=jnp.float32)
    z = z.astype(max_dtype)
    m = jnp.maximum(
        jnp.maximum(z[:, :cout], z[:, cout:2 * cout]),
        jnp.maximum(z[:, 2 * cout:3 * cout], z[:, 3 * cout:]),
    )
    y = jnp.maximum(m + b_ref[...].astype(max_dtype), 0.0).astype(out_dtype)
    o_ref[...] = y.reshape(o_ref.shape)


def _fc_kernel(a_ref, w1_ref, b1_ref, w2_ref, b2_ref, o_ref):
    h = jnp.dot(a_ref[...], w1_ref[...], preferred_element_type=jnp.float32)
    h = jnp.maximum(h + b1_ref[...], 0.0).astype(_BF)
    o_ref[...] = (jnp.dot(h, w2_ref[...], preferred_element_type=jnp.float32)
                  + b2_ref[...])


def _conv_layer(rwins, wp, b, tb, hs, ws, cin, cout, max_dtype, out_dtype):
    bsz = rwins[0].shape[0]
    tb = min(tb, bsz)
    dt = rwins[0].dtype
    return pl.pallas_call(
        functools.partial(_conv_win_kernel, hs=hs, ws=ws, cout=cout,
                          max_dtype=max_dtype, out_dtype=out_dtype),
        out_shape=jax.ShapeDtypeStruct((bsz, hs, ws, cout), out_dtype),
        grid=(bsz // tb,),
        in_specs=[pl.BlockSpec((tb, hs + 2, ws, 4 * cin), lambda i: (i, 0, 0, 0))
                  for _ in range(2)]
                 + [pl.BlockSpec((4, 4 * cin, 4 * cout), lambda i: (0, 0, 0)),
                    pl.BlockSpec((1, cout), lambda i: (0, 0))],
        out_specs=pl.BlockSpec((tb, hs, ws, cout), lambda i: (i, 0, 0, 0)),
        compiler_params=pltpu.CompilerParams(
            dimension_semantics=("arbitrary",)),
    )(rwins[0].astype(dt), rwins[1].astype(dt), wp, b)


def kernel(x_nchw, w1g, b1, w2g, b2, w3g, b3, fc1_wg, fc1_b, fc2_wg, fc2_b):
    bsz = x_nchw.shape[0]
    x = x_nchw.reshape(bsz, 64, 64)

    xp = jnp.pad(x, ((0, 0), (1, 1), (1, 1)))
    x16 = jnp.stack([
        xp[:, sy:sy + 63:2, sx:sx + 63:2]
        for sy in range(4) for sx in range(4)
    ], axis=-1)
    x16 = x16.astype(_BF).reshape(bsz * 1024, 16)
    wp1 = _pack_patch_weights(w1g, 1, 32).reshape(16, 128).astype(_BF)
    b1b = b1.astype(_BF)

    m1 = bsz * 1024
    tm = min(32768, m1)
    y1 = pl.pallas_call(
        _l1_kernel,
        out_shape=jax.ShapeDtypeStruct((m1, 32), _BF),
        grid=(m1 // tm,),
        in_specs=[pl.BlockSpec((tm, 16), lambda i: (i, 0)),
                  pl.BlockSpec((16, 128), lambda i: (0, 0)),
                  pl.BlockSpec((1, 32), lambda i: (0, 0))],
        out_specs=pl.BlockSpec((tm, 32), lambda i: (i, 0)),
        compiler_params=pltpu.CompilerParams(
            dimension_semantics=("arbitrary",)),
    )(x16, wp1, b1b)
    y1 = y1.reshape(bsz, 32, 32, 32)

    wp2 = _pack_patch_weights(w2g, 32, 64).astype(_BF)
    y2 = _conv_layer(_row_windows(y1, _BF), wp2, b2, 64, 16, 16, 32, 64,
                     _BF, _BF)

    wp3 = _pack_patch_weights(w3g, 64, 128)
    y3 = _conv_layer(_row_windows(y2, _BF), wp3, b3, 64, 8, 8, 64, 128,
                     jnp.float32, _BF)

    feat = y3.reshape(bsz, 8 * 8 * 128)
    nc = fc2_wg.shape[1]
    ncp = max(128, ((nc + 127) // 128) * 128)
    w1f = fc1_wg.astype(_BF)
    w2f = jnp.pad(fc2_wg, ((0, 0), (0, ncp - nc))).astype(_BF)
    b2f = jnp.pad(fc2_b, ((0, 0), (0, ncp - nc)))
    tmf = min(256, bsz)
    out = pl.pallas_call(
        _fc_kernel,
        out_shape=jax.ShapeDtypeStruct((bsz, ncp), jnp.float32),
        grid=(bsz // tmf,),
        in_specs=[pl.BlockSpec((tmf, 8192), lambda i: (i, 0)),
                  pl.BlockSpec((8192, 512), lambda i: (0, 0)),
                  pl.BlockSpec((1, 512), lambda i: (0, 0)),
                  pl.BlockSpec((512, ncp), lambda i: (0, 0)),
                  pl.BlockSpec((1, ncp), lambda i: (0, 0))],
        out_specs=pl.BlockSpec((tmf, ncp), lambda i: (i, 0)),
        compiler_params=pltpu.CompilerParams(
            dimension_semantics=("arbitrary",)),
    )(feat, w1f, fc1_b, w2f, b2f)
    return out[:, :nc]
